# baseline (speedup 1.0000x reference)
import functools

import jax
import jax.numpy as jnp
import numpy as np
from jax.experimental import pallas as pl
from jax.experimental.pallas import tpu as pltpu
from jax.sharding import Mesh, PartitionSpec as P

BN_EPS = 1e-5
LEAKY_SLOPE = 0.01

KB1 = 512
BM = 2048
BN = 512


def _sums_kernel(x_ref, s1_ref, s2_ref):
    x = x_ref[...]
    s1_ref[...] = jnp.sum(x, axis=0, keepdims=True)
    s2_ref[...] = jnp.sum(x * x, axis=0, keepdims=True)


def _bn_kernel(n_total, x_ref, s1_ref, s2_ref, gamma_ref, beta_ref, h_ref):
    inv_n = 1.0 / n_total
    mean = jnp.sum(s1_ref[...], axis=0, keepdims=True) * inv_n
    ex2 = jnp.sum(s2_ref[...], axis=0, keepdims=True) * inv_n
    var = ex2 - mean * mean
    s = gamma_ref[...] * jax.lax.rsqrt(var + BN_EPS)
    t = beta_ref[...] - mean * s
    h_ref[...] = (x_ref[...] * s + t).astype(jnp.bfloat16)


def _mm_kernel(h_ref, w_ref, b_ref, o_ref):
    acc = jax.lax.dot_general(
        h_ref[...], w_ref[...],
        dimension_numbers=(((1,), (1,)), ((), ())),
        preferred_element_type=jnp.float32,
    )
    y = acc + b_ref[...]
    o_ref[...] = jnp.where(y >= 0.0, y, LEAKY_SLOPE * y)


def _pipeline(n_total, x_loc, gamma2, beta2, W16, b2):
    b_loc, d_in = x_loc.shape
    d_out = W16.shape[0]

    s1, s2 = pl.pallas_call(
        _sums_kernel,
        grid=(d_in // KB1,),
        in_specs=[pl.BlockSpec((b_loc, KB1), lambda k: (0, k))],
        out_specs=[
            pl.BlockSpec((1, KB1), lambda k: (0, k)),
            pl.BlockSpec((1, KB1), lambda k: (0, k)),
        ],
        out_shape=[
            jax.ShapeDtypeStruct((1, d_in), jnp.float32),
            jax.ShapeDtypeStruct((1, d_in), jnp.float32),
        ],
    )(x_loc)

    s1_all = jax.lax.all_gather(s1, 'b', axis=0, tiled=True)
    s2_all = jax.lax.all_gather(s2, 'b', axis=0, tiled=True)

    nd = s1_all.shape[0]
    h = pl.pallas_call(
        functools.partial(_bn_kernel, float(n_total)),
        grid=(d_in // KB1,),
        in_specs=[
            pl.BlockSpec((b_loc, KB1), lambda k: (0, k)),
            pl.BlockSpec((nd, KB1), lambda k: (0, k)),
            pl.BlockSpec((nd, KB1), lambda k: (0, k)),
            pl.BlockSpec((1, KB1), lambda k: (0, k)),
            pl.BlockSpec((1, KB1), lambda k: (0, k)),
        ],
        out_specs=pl.BlockSpec((b_loc, KB1), lambda k: (0, k)),
        out_shape=jax.ShapeDtypeStruct((b_loc, d_in), jnp.bfloat16),
    )(x_loc, s1_all, s2_all, gamma2, beta2)

    out = pl.pallas_call(
        _mm_kernel,
        grid=(b_loc // BM, d_out // BN),
        in_specs=[
            pl.BlockSpec((BM, d_in), lambda m, n: (m, 0)),
            pl.BlockSpec((BN, d_in), lambda m, n: (n, 0)),
            pl.BlockSpec((1, BN), lambda m, n: (0, n)),
        ],
        out_specs=pl.BlockSpec((BM, BN), lambda m, n: (m, n)),
        out_shape=jax.ShapeDtypeStruct((b_loc, d_out), jnp.float32),
    )(h, W16, b2)
    return out


@functools.partial(jax.jit, donate_argnums=())
def kernel(x, gamma, beta, W, b):
    B, IN = x.shape
    OUT = W.shape[0]

    gamma2 = gamma.reshape(1, IN)
    beta2 = beta.reshape(1, IN)
    b2 = b.reshape(1, OUT)
    W16 = W.astype(jnp.bfloat16)

    devs = jax.devices()
    nd = 2 if len(devs) >= 2 and B % (2 * BM) == 0 else 1
    mesh = Mesh(np.array(devs[:nd]), ('b',))

    fn = jax.shard_map(
        functools.partial(_pipeline, B),
        mesh=mesh,
        in_specs=(P('b', None), P(None, None), P(None, None),
                  P(None, None), P(None, None)),
        out_specs=P('b', None),
        check_vma=False,
    )
    return fn(x, gamma2, beta2, W16, b2)


# speedup vs baseline: 2.0088x; 2.0088x over previous
import functools

import jax
import jax.numpy as jnp
from jax.experimental import pallas as pl
from jax.experimental.pallas import tpu as pltpu

BN_EPS = 1e-5
LEAKY_SLOPE = 0.01

KB1 = 512
BM = 1024
BN = 1024


def _bn_kernel(x_ref, gamma_ref, beta_ref, h_ref):
    x = x_ref[...]
    n = x.shape[0]
    mean = jnp.sum(x, axis=0, keepdims=True) * (1.0 / n)
    ex2 = jnp.sum(x * x, axis=0, keepdims=True) * (1.0 / n)
    var = ex2 - mean * mean
    s = gamma_ref[...] * jax.lax.rsqrt(var + BN_EPS)
    t = beta_ref[...] - mean * s
    h_ref[...] = (x * s + t).astype(jnp.bfloat16)


def _mm_kernel(h_ref, w_ref, b_ref, o_ref):
    acc = jax.lax.dot_general(
        h_ref[...], w_ref[...],
        dimension_numbers=(((1,), (1,)), ((), ())),
        preferred_element_type=jnp.float32,
    )
    y = acc + b_ref[...]
    o_ref[...] = jnp.where(y >= 0.0, y, LEAKY_SLOPE * y)


@functools.partial(jax.jit, donate_argnums=())
def kernel(x, gamma, beta, W, b):
    B, IN = x.shape
    OUT = W.shape[0]

    gamma2 = gamma.reshape(1, IN)
    beta2 = beta.reshape(1, IN)
    b2 = b.reshape(1, OUT)
    W16 = W.astype(jnp.bfloat16)

    h = pl.pallas_call(
        _bn_kernel,
        grid=(IN // KB1,),
        in_specs=[
            pl.BlockSpec((B, KB1), lambda k: (0, k)),
            pl.BlockSpec((1, KB1), lambda k: (0, k)),
            pl.BlockSpec((1, KB1), lambda k: (0, k)),
        ],
        out_specs=pl.BlockSpec((B, KB1), lambda k: (0, k)),
        out_shape=jax.ShapeDtypeStruct((B, IN), jnp.bfloat16),
        compiler_params=pltpu.CompilerParams(
            dimension_semantics=("parallel",),
        ),
    )(x, gamma2, beta2)

    out = pl.pallas_call(
        _mm_kernel,
        grid=(B // BM, OUT // BN),
        in_specs=[
            pl.BlockSpec((BM, IN), lambda m, n: (m, 0)),
            pl.BlockSpec((BN, IN), lambda m, n: (n, 0)),
            pl.BlockSpec((1, BN), lambda m, n: (0, n)),
        ],
        out_specs=pl.BlockSpec((BM, BN), lambda m, n: (m, n)),
        out_shape=jax.ShapeDtypeStruct((B, OUT), jnp.float32),
        compiler_params=pltpu.CompilerParams(
            dimension_semantics=("parallel", "arbitrary"),
        ),
    )(h, W16, b2)
    return out
